# baseline (speedup 1.0000x reference)
import functools

import jax
import jax.numpy as jnp
from jax import lax
from jax.experimental import pallas as pl
from jax.experimental.pallas import tpu as pltpu
from jax.experimental.pallas import tpu_sc as plsc

N = 10000
E = 320000
NFEAT = 128
H1 = 8
DH1 = 16
D1 = H1 * DH1
DH = 64
D2 = 64
NBASE = 4

NC = 2
NS = 16
NW = NC * NS
EW = E // NW
K = 128
CH = -(-EW // K)
EWP = CH * K
RPT = 632
NP = RPT * NS

f32 = jnp.float32
i32 = jnp.int32

_params = pltpu.CompilerParams(use_tc_tiling_on_sc=False,
                               needs_layout_passes=False)


def _mesh():
    return plsc.VectorSubcoreMesh(core_axis_name="c", subcore_axis_name="s")


_SPLAT_DNUMS = lax.GatherDimensionNumbers(
    offset_dims=(), collapsed_slice_dims=(0,), start_index_map=(0,))


def _splat(vec, h):
    idx = jnp.full((16, 1), h, dtype=i32)
    return lax.gather(vec, idx, _SPLAT_DNUMS, (1,),
                      mode=lax.GatherScatterMode.PROMISE_IN_BOUNDS)


def _wid_r0():
    c = lax.axis_index("c")
    s = lax.axis_index("s")
    return c, s, c * NS + s, s * RPT


@functools.partial(
    pl.kernel,
    out_type=jax.ShapeDtypeStruct((NC, NP, 16), f32),
    mesh=_mesh(),
    compiler_params=_params,
    scratch_types=[
        pltpu.VMEM((CH, K), i32),
        pltpu.VMEM((K, 16), f32),
        pltpu.VMEM_SHARED((NP, 16), f32),
    ],
)
def _deg_kernel(dst3, zeros16, ones16, out, dst_v, ones_v, acc):
    c, s, wid, r0 = _wid_r0()
    pltpu.sync_copy(zeros16.at[pl.ds(r0, RPT)], acc.at[pl.ds(r0, RPT)])
    pltpu.sync_copy(dst3.at[wid], dst_v)
    pltpu.sync_copy(ones16, ones_v)
    plsc.subcore_barrier()

    @pl.loop(0, CH)
    def _chunk(j):
        pltpu.sync_copy(ones_v, acc.at[dst_v.at[j]], add=True)

    plsc.subcore_barrier()
    pltpu.sync_copy(acc.at[pl.ds(r0, RPT)], out.at[c, pl.ds(r0, RPT)])


def _make_prop(D):
    @functools.partial(
        pl.kernel,
        out_type=jax.ShapeDtypeStruct((NC, NP, D), f32),
        mesh=_mesh(),
        compiler_params=_params,
        scratch_types=[
            pltpu.VMEM((CH, K), i32),
            pltpu.VMEM((CH, K), i32),
            pltpu.VMEM((K, D), f32),
            pltpu.VMEM_SHARED((NP, D), f32),
            pltpu.SemaphoreType.DMA,
        ],
    )
    def prop(src3, dst3, z, zerosD, out, src_v, dst_v, rows_v, acc, sem):
        c, s, wid, r0 = _wid_r0()
        pltpu.sync_copy(zerosD.at[pl.ds(r0, RPT)], acc.at[pl.ds(r0, RPT)])
        pltpu.sync_copy(src3.at[wid], src_v)
        pltpu.sync_copy(dst3.at[wid], dst_v)
        plsc.subcore_barrier()

        @pl.loop(0, CH)
        def _chunk(j):
            pltpu.async_copy(z.at[src_v.at[j]], rows_v, sem).wait()
            pltpu.sync_copy(rows_v, acc.at[dst_v.at[j]], add=True)

        plsc.subcore_barrier()
        pltpu.sync_copy(acc.at[pl.ds(r0, RPT)], out.at[c, pl.ds(r0, RPT)])

    return prop


_prop64 = _make_prop(DH)


@functools.partial(
    pl.kernel,
    out_type=(
        jax.ShapeDtypeStruct((NW, EWP, 16), f32),
        jax.ShapeDtypeStruct((NC, NP, 16), f32),
    ),
    mesh=_mesh(),
    compiler_params=_params,
    scratch_types=[
        pltpu.VMEM((CH, K), i32),
        pltpu.VMEM((CH, K), i32),
        pltpu.VMEM((K, 16), f32),
        pltpu.VMEM((K, 16), f32),
        pltpu.VMEM((K, 16), f32),
        pltpu.VMEM_SHARED((NP, 16), f32),
        pltpu.SemaphoreType.DMA,
    ],
)
def _attp_kernel(src3, dst3, ta, tb, zeros16, pout, outd,
                 src_v, dst_v, a_v, b_v, p_v, den, sem):
    c, s, wid, r0 = _wid_r0()
    pltpu.sync_copy(zeros16.at[pl.ds(r0, RPT)], den.at[pl.ds(r0, RPT)])
    pltpu.sync_copy(src3.at[wid], src_v)
    pltpu.sync_copy(dst3.at[wid], dst_v)
    plsc.subcore_barrier()
    lane = lax.iota(i32, 16)
    mask8 = lane < 8

    @pl.loop(0, CH)
    def _chunk(j):
        pltpu.async_copy(ta.at[src_v.at[j]], a_v, sem).wait()
        pltpu.async_copy(tb.at[dst_v.at[j]], b_v, sem).wait()

        @pl.loop(0, K, unroll=8)
        def _edge(e):
            t = a_v[e, :] + b_v[e, :]
            t = jnp.where(t > 0.0, t, t * 0.2)
            pv = jnp.exp(t)
            p_v[e, :] = jnp.where(mask8, pv, 0.0)

        pltpu.sync_copy(p_v, pout.at[wid, pl.ds(j * K, K)])
        pltpu.sync_copy(p_v, den.at[dst_v.at[j]], add=True)

    plsc.subcore_barrier()
    pltpu.sync_copy(den.at[pl.ds(r0, RPT)], outd.at[c, pl.ds(r0, RPT)])


def _make_att1(h0):
    @functools.partial(
        pl.kernel,
        out_type=jax.ShapeDtypeStruct((NC, NP, DH), f32),
        mesh=_mesh(),
        compiler_params=_params,
        scratch_types=[
            pltpu.VMEM((CH, K), i32),
            pltpu.VMEM((CH, K), i32),
            pltpu.VMEM((K, DH), f32),
            pltpu.VMEM((K, 16), f32),
            pltpu.VMEM_SHARED((NP, DH), f32),
            pltpu.SemaphoreType.DMA,
        ],
    )
    def att1(src3, dst3, zh, p1, zerosH, out,
             src_v, dst_v, rows_v, p_v, acc, sem):
        c, s, wid, r0 = _wid_r0()
        pltpu.sync_copy(zerosH.at[pl.ds(r0, RPT)], acc.at[pl.ds(r0, RPT)])
        pltpu.sync_copy(src3.at[wid], src_v)
        pltpu.sync_copy(dst3.at[wid], dst_v)
        plsc.subcore_barrier()

        @pl.loop(0, CH)
        def _chunk(j):
            pltpu.async_copy(zh.at[src_v.at[j]], rows_v, sem).wait()
            pltpu.sync_copy(p1.at[wid, pl.ds(j * K, K)], p_v)

            @pl.loop(0, K, unroll=8)
            def _edge(e):
                pv = p_v[e, :]
                for h in range(4):
                    sp = _splat(pv, h0 + h)
                    sl = pl.ds(h * DH1, DH1)
                    rows_v[e, sl] = rows_v[e, sl] * sp

            pltpu.sync_copy(rows_v, acc.at[dst_v.at[j]], add=True)

        plsc.subcore_barrier()
        pltpu.sync_copy(acc.at[pl.ds(r0, RPT)], out.at[c, pl.ds(r0, RPT)])

    return att1


_att1a = _make_att1(0)
_att1b = _make_att1(4)


@functools.partial(
    pl.kernel,
    out_type=(
        jax.ShapeDtypeStruct((NW, EWP, 16), f32),
        jax.ShapeDtypeStruct((NC, NP, 16), f32),
    ),
    mesh=_mesh(),
    compiler_params=_params,
    scratch_types=[
        pltpu.VMEM((CH, K), i32),
        pltpu.VMEM((CH, K), i32),
        pltpu.VMEM((NP,), f32),
        pltpu.VMEM((NP,), f32),
        pltpu.VMEM((K,), f32),
        pltpu.VMEM((K, 16), f32),
        pltpu.VMEM_SHARED((NP, 16), f32),
    ],
)
def _attp2_kernel(src3, dst3, es2, ed2, zeros16, pout, outd,
                  src_v, dst_v, es_v, ed_v, pg_v, p_v, den):
    c, s, wid, r0 = _wid_r0()
    pltpu.sync_copy(zeros16.at[pl.ds(r0, RPT)], den.at[pl.ds(r0, RPT)])
    pltpu.sync_copy(src3.at[wid], src_v)
    pltpu.sync_copy(dst3.at[wid], dst_v)
    pltpu.sync_copy(es2, es_v)
    pltpu.sync_copy(ed2, ed_v)
    plsc.subcore_barrier()

    @pl.loop(0, CH)
    def _chunk(j):
        @pl.loop(0, K // 16, unroll=8)
        def _group(m):
            srcs = src_v[j, pl.ds(m * 16, 16)]
            dsts = dst_v[j, pl.ds(m * 16, 16)]
            t = plsc.load_gather(es_v, [srcs]) + plsc.load_gather(ed_v, [dsts])
            t = jnp.where(t > 0.0, t, t * 0.2)
            pg_v[pl.ds(m * 16, 16)] = jnp.exp(t)

        @pl.loop(0, K, unroll=8)
        def _edge(e):
            p_v[e, :] = plsc.load_gather(pg_v, [jnp.full((16,), e, i32)])

        pltpu.sync_copy(p_v, pout.at[wid, pl.ds(j * K, K)])
        pltpu.sync_copy(p_v, den.at[dst_v.at[j]], add=True)

    plsc.subcore_barrier()
    pltpu.sync_copy(den.at[pl.ds(r0, RPT)], outd.at[c, pl.ds(r0, RPT)])


@functools.partial(
    pl.kernel,
    out_type=jax.ShapeDtypeStruct((NC, NP, D2), f32),
    mesh=_mesh(),
    compiler_params=_params,
    scratch_types=[
        pltpu.VMEM((CH, K), i32),
        pltpu.VMEM((CH, K), i32),
        pltpu.VMEM((K, D2), f32),
        pltpu.VMEM((K, 16), f32),
        pltpu.VMEM_SHARED((NP, D2), f32),
        pltpu.SemaphoreType.DMA,
    ],
)
def _att2_kernel(src3, dst3, z2p, p2, zeros64, out,
                 src_v, dst_v, rows_v, p_v, acc, sem):
    c, s, wid, r0 = _wid_r0()
    pltpu.sync_copy(zeros64.at[pl.ds(r0, RPT)], acc.at[pl.ds(r0, RPT)])
    pltpu.sync_copy(src3.at[wid], src_v)
    pltpu.sync_copy(dst3.at[wid], dst_v)
    plsc.subcore_barrier()

    @pl.loop(0, CH)
    def _chunk(j):
        pltpu.async_copy(z2p.at[src_v.at[j]], rows_v, sem).wait()
        pltpu.sync_copy(p2.at[wid, pl.ds(j * K, K)], p_v)

        @pl.loop(0, K, unroll=8)
        def _edge(e):
            sp = p_v[e, :]
            for q in range(D2 // 16):
                sl = pl.ds(q * 16, 16)
                rows_v[e, sl] = rows_v[e, sl] * sp

        pltpu.sync_copy(rows_v, acc.at[dst_v.at[j]], add=True)

    plsc.subcore_barrier()
    pltpu.sync_copy(acc.at[pl.ds(r0, RPT)], out.at[c, pl.ds(r0, RPT)])


BLK = NP // 8
GRID = NP // BLK


def _rows(width):
    return pl.BlockSpec((BLK, width), lambda i: (i, 0))


def _prows(width):
    return pl.BlockSpec((2, BLK, width), lambda i: (0, i, 0))


def _full(shape):
    return pl.BlockSpec(shape, lambda i: tuple(0 for _ in shape))


def _inv_from_deg(dega_ref):
    deg = dega_ref[0, :, :1] + dega_ref[1, :, :1]
    base = pl.program_id(0) * BLK
    rowmask = base + lax.broadcasted_iota(i32, (BLK, 1), 0) < N
    return jnp.where(jnp.logical_and(rowmask, deg > 0.0),
                     lax.rsqrt(jnp.maximum(deg, 1.0)), 0.0)


def _t1_body(x_ref, br1_ref, s1_ref, dega_ref, za_ref, zb_ref):
    inv = _inv_from_deg(dega_ref)
    wc = jnp.dot(br1_ref[...], s1_ref[...], preferred_element_type=f32)
    z = jnp.dot(x_ref[...], wc, preferred_element_type=f32) * inv
    za_ref[...] = z[:, :DH]
    zb_ref[...] = z[:, DH:]


_t1 = pl.pallas_call(
    _t1_body,
    grid=(GRID,),
    in_specs=[_rows(NFEAT), _full((NFEAT, NBASE * DH1)),
              _full((NBASE * DH1, D1)), _prows(16)],
    out_specs=(_rows(DH), _rows(DH)),
    out_shape=(jax.ShapeDtypeStruct((NP, DH), f32),
               jax.ShapeDtypeStruct((NP, DH), f32)))


def _t2_body(acca_ref, accb_ref, dega_ref, asel_ref, bsel_ref,
             z1a_ref, z1b_ref, a_ref, b_ref):
    inv = _inv_from_deg(dega_ref)
    z1a = (acca_ref[0] + acca_ref[1]) * inv
    z1b = (accb_ref[0] + accb_ref[1]) * inv
    z1a_ref[...] = z1a
    z1b_ref[...] = z1b
    a_ref[...] = (jnp.dot(z1a, asel_ref[:DH, :], preferred_element_type=f32)
                  + jnp.dot(z1b, asel_ref[DH:, :],
                            preferred_element_type=f32))
    b_ref[...] = (jnp.dot(z1a, bsel_ref[:DH, :], preferred_element_type=f32)
                  + jnp.dot(z1b, bsel_ref[DH:, :],
                            preferred_element_type=f32))


_t2 = pl.pallas_call(
    _t2_body,
    grid=(GRID,),
    in_specs=[_prows(DH), _prows(DH), _prows(16),
              _full((D1, 16)), _full((D1, 16))],
    out_specs=(_rows(DH), _rows(DH), _rows(16), _rows(16)),
    out_shape=(jax.ShapeDtypeStruct((NP, DH), f32),
               jax.ShapeDtypeStruct((NP, DH), f32),
               jax.ShapeDtypeStruct((NP, 16), f32),
               jax.ShapeDtypeStruct((NP, 16), f32)))


def _t3_body(outa_ref, outb_ref, outd_ref, dega_ref, bias_ref, br2_ref,
             s2sel_ref, erep_ref, z2pre_ref):
    inv = _inv_from_deg(dega_ref)
    den = outd_ref[0] + outd_ref[1]
    div = jnp.dot(den[:, :H1], erep_ref[...], preferred_element_type=f32)
    num_a = outa_ref[0] + outa_ref[1]
    num_b = outb_ref[0] + outb_ref[1]
    ha = num_a / (div[:, :DH] + 1e-16) + bias_ref[:, :DH]
    hb = num_b / (div[:, DH:] + 1e-16) + bias_ref[:, DH:]
    ha = jnp.where(ha > 0.0, ha, jnp.exp(ha) - 1.0)
    hb = jnp.where(hb > 0.0, hb, jnp.exp(hb) - 1.0)
    wc2 = jnp.dot(br2_ref[...], s2sel_ref[...], preferred_element_type=f32)
    z2 = (jnp.dot(ha, wc2[:DH, :], preferred_element_type=f32)
          + jnp.dot(hb, wc2[DH:, :], preferred_element_type=f32))
    z2pre_ref[...] = z2 * inv


_t3 = pl.pallas_call(
    _t3_body,
    grid=(GRID,),
    in_specs=[_prows(DH), _prows(DH), _prows(16), _prows(16),
              _full((1, D1)), _full((D1, NBASE * D2)),
              _full((NBASE * D2, D2)), _full((H1, D1))],
    out_specs=_rows(D2),
    out_shape=jax.ShapeDtypeStruct((NP, D2), f32))


def _t4_body(acc_ref, dega_ref, a2c_ref, z2p_ref, s2_ref):
    inv = _inv_from_deg(dega_ref)
    z2p = (acc_ref[0] + acc_ref[1]) * inv
    z2p_ref[...] = z2p
    s2_ref[...] = jnp.dot(z2p, a2c_ref[...], preferred_element_type=f32)


_t4 = pl.pallas_call(
    _t4_body,
    grid=(GRID,),
    in_specs=[_prows(D2), _prows(16), _full((D2, 2))],
    out_specs=(_rows(D2), _rows(2)),
    out_shape=(jax.ShapeDtypeStruct((NP, D2), f32),
               jax.ShapeDtypeStruct((NP, 2), f32)))


def _t5_body(outp_ref, outd_ref, fin_ref):
    num = outp_ref[0] + outp_ref[1]
    den = outd_ref[0, :, :1] + outd_ref[1, :, :1]
    fin_ref[...] = num / (den + 1e-16)


_t5 = pl.pallas_call(
    _t5_body,
    grid=(GRID,),
    in_specs=[_prows(D2), _prows(16)],
    out_specs=_rows(D2),
    out_shape=jax.ShapeDtypeStruct((N, D2), f32))


def kernel(x, edge_index, basis1, coef1, a1_src, a1_dst, bias1,
           basis2, coef2, a2_src, a2_dst):
    src = edge_index[0].astype(i32)
    dst = edge_index[1].astype(i32)
    src3 = jnp.concatenate(
        [src.reshape(NW, EW), jnp.zeros((NW, EWP - EW), i32)],
        axis=1).reshape(NW, CH, K)
    dst3 = jnp.concatenate(
        [dst.reshape(NW, EW), jnp.full((NW, EWP - EW), N, i32)],
        axis=1).reshape(NW, CH, K)

    xp = jnp.pad(x, ((0, NP - N), (0, 0)))
    br1 = jnp.transpose(basis1, (1, 0, 2)).reshape(NFEAT, NBASE * DH1)
    s1 = jnp.kron(coef1.T, jnp.eye(DH1, dtype=f32))
    eyeh = jnp.eye(H1, dtype=f32)
    asel_l = (a1_src[:, :, None] * eyeh[:, None, :]).reshape(D1, H1)
    asel_r = (a1_dst[:, :, None] * eyeh[:, None, :]).reshape(D1, H1)
    ta_sel = jnp.concatenate([asel_l, asel_r], axis=1)
    tb_sel = jnp.concatenate([asel_r, asel_l], axis=1)
    br2 = jnp.transpose(basis2, (1, 0, 2)).reshape(D1, NBASE * D2)
    s2sel = jnp.kron(coef2.T, jnp.eye(D2, dtype=f32))
    erep = (eyeh[:, :, None] * jnp.ones((1, 1, DH1), f32)).reshape(H1, D1)
    a2cat = jnp.stack([a2_src[0], a2_dst[0]], axis=1)

    zeros64 = jnp.zeros((NP, DH), f32)
    zeros16 = jnp.zeros((NP, 16), f32)
    ones16 = jnp.ones((K, 16), f32)

    dega = _deg_kernel(dst3, zeros16, ones16)
    zpa, zpb = _t1(xp, br1, s1, dega)
    acca = _prop64(src3, dst3, zpa, zeros64)
    accb = _prop64(src3, dst3, zpb, zeros64)
    z1a, z1b, ta, tb = _t2(acca, accb, dega, ta_sel, tb_sel)
    p1, outd1 = _attp_kernel(src3, dst3, ta, tb, zeros16)
    outa = _att1a(src3, dst3, z1a, p1, zeros64)
    outb = _att1b(src3, dst3, z1b, p1, zeros64)
    z2pre = _t3(outa, outb, outd1, dega, bias1.reshape(1, D1), br2, s2sel,
                erep)
    acc2 = _prop64(src3, dst3, z2pre, zeros64)
    z2p, s2 = _t4(acc2, dega, a2cat)
    es2 = s2[:, 0]
    ed2 = s2[:, 1]
    p2, outd2 = _attp2_kernel(src3, dst3, es2, ed2, zeros16)
    out2 = _att2_kernel(src3, dst3, z2p, p2, zeros64)
    return _t5(out2, outd2)


# speedup vs baseline: 1.0957x; 1.0957x over previous
import functools

import jax
import jax.numpy as jnp
from jax import lax
from jax.experimental import pallas as pl
from jax.experimental.pallas import tpu as pltpu
from jax.experimental.pallas import tpu_sc as plsc

N = 10000
E = 320000
NFEAT = 128
H1 = 8
DH1 = 16
D1 = H1 * DH1
DH = 64
D2 = 64
NBASE = 4

NC = 2
NS = 16
NW = NC * NS
EW = E // NW
K = 128
CH = 80
EWP = CH * K
RPT = 632
NP = RPT * NS

f32 = jnp.float32
i32 = jnp.int32

_params = pltpu.CompilerParams(use_tc_tiling_on_sc=False,
                               needs_layout_passes=False)


def _mesh():
    return plsc.VectorSubcoreMesh(core_axis_name="c", subcore_axis_name="s")


_SPLAT_DNUMS = lax.GatherDimensionNumbers(
    offset_dims=(), collapsed_slice_dims=(0,), start_index_map=(0,))


def _splat(vec, h):
    idx = jnp.full((16, 1), h, dtype=i32)
    return lax.gather(vec, idx, _SPLAT_DNUMS, (1,),
                      mode=lax.GatherScatterMode.PROMISE_IN_BOUNDS)


def _wid_r0():
    c = lax.axis_index("c")
    s = lax.axis_index("s")
    return c, s, c * NS + s, s * RPT


@functools.partial(
    pl.kernel,
    out_type=jax.ShapeDtypeStruct((NC, NP, 16), f32),
    mesh=_mesh(),
    compiler_params=_params,
    scratch_types=[
        pltpu.VMEM((CH, K), i32),
        pltpu.VMEM((K, 16), f32),
        pltpu.VMEM_SHARED((NP, 16), f32),
    ],
)
def _deg_kernel(dst3, zeros16, ones16, out, dst_v, ones_v, acc):
    c, s, wid, r0 = _wid_r0()
    pltpu.sync_copy(zeros16.at[pl.ds(r0, RPT)], acc.at[pl.ds(r0, RPT)])
    pltpu.sync_copy(dst3.at[wid], dst_v)
    pltpu.sync_copy(ones16, ones_v)
    plsc.subcore_barrier()

    @pl.loop(0, CH)
    def _chunk(j):
        pltpu.sync_copy(ones_v, acc.at[dst_v.at[j]], add=True)

    plsc.subcore_barrier()
    pltpu.sync_copy(acc.at[pl.ds(r0, RPT)], out.at[c, pl.ds(r0, RPT)])


def _make_prop(D):
    @functools.partial(
        pl.kernel,
        out_type=jax.ShapeDtypeStruct((NC, NP, D), f32),
        mesh=_mesh(),
        compiler_params=_params,
        scratch_types=[
            pltpu.VMEM((CH, K), i32),
            pltpu.VMEM((CH, K), i32),
            pltpu.VMEM((2, K, D), f32),
            pltpu.VMEM_SHARED((NP, D), f32),
            pltpu.SemaphoreType.DMA,
        ],
    )
    def prop(src3, dst3, z, zerosD, out, src_v, dst_v, rows_v, acc, sem):
        c, s, wid, r0 = _wid_r0()
        pltpu.sync_copy(zerosD.at[pl.ds(r0, RPT)], acc.at[pl.ds(r0, RPT)])
        pltpu.sync_copy(src3.at[wid], src_v)
        pltpu.sync_copy(dst3.at[wid], dst_v)
        plsc.subcore_barrier()
        pltpu.async_copy(z.at[src_v.at[0]], rows_v.at[0], sem)

        @pl.loop(0, CH, step=2)
        def _chunk(j):
            for b in range(2):
                jj = j + b

                @pl.when(jj + 1 < CH)
                def _pref():
                    pltpu.async_copy(z.at[src_v.at[jj + 1]],
                                     rows_v.at[1 - b], sem)

                pltpu.make_async_copy(z.at[src_v.at[jj]],
                                      rows_v.at[b], sem).wait()
                pltpu.sync_copy(rows_v.at[b], acc.at[dst_v.at[jj]], add=True)

        plsc.subcore_barrier()
        pltpu.sync_copy(acc.at[pl.ds(r0, RPT)], out.at[c, pl.ds(r0, RPT)])

    return prop


_prop64 = _make_prop(DH)


@functools.partial(
    pl.kernel,
    out_type=(
        jax.ShapeDtypeStruct((NW, EWP, 16), f32),
        jax.ShapeDtypeStruct((NC, NP, 16), f32),
    ),
    mesh=_mesh(),
    compiler_params=_params,
    scratch_types=[
        pltpu.VMEM((CH, K), i32),
        pltpu.VMEM((CH, K), i32),
        pltpu.VMEM((2, K, 16), f32),
        pltpu.VMEM((2, K, 16), f32),
        pltpu.VMEM((2, K, 16), f32),
        pltpu.VMEM_SHARED((NP, 16), f32),
        pltpu.SemaphoreType.DMA,
    ],
)
def _attp_kernel(src3, dst3, ta, tb, zeros16, pout, outd,
                 src_v, dst_v, a_v, b_v, p_v, den, sem):
    c, s, wid, r0 = _wid_r0()
    pltpu.sync_copy(zeros16.at[pl.ds(r0, RPT)], den.at[pl.ds(r0, RPT)])
    pltpu.sync_copy(src3.at[wid], src_v)
    pltpu.sync_copy(dst3.at[wid], dst_v)
    plsc.subcore_barrier()
    lane = lax.iota(i32, 16)
    mask8 = lane < 8
    pltpu.async_copy(ta.at[src_v.at[0]], a_v.at[0], sem)
    pltpu.async_copy(tb.at[dst_v.at[0]], b_v.at[0], sem)

    @pl.loop(0, CH, step=2)
    def _chunk(j):
        for b in range(2):
            jj = j + b

            @pl.when(jj + 1 < CH)
            def _pref():
                pltpu.async_copy(ta.at[src_v.at[jj + 1]], a_v.at[1 - b], sem)
                pltpu.async_copy(tb.at[dst_v.at[jj + 1]], b_v.at[1 - b], sem)

            pltpu.make_async_copy(ta.at[src_v.at[jj]], a_v.at[b], sem).wait()
            pltpu.make_async_copy(tb.at[dst_v.at[jj]], b_v.at[b], sem).wait()

            @pl.loop(0, K)
            def _edge(e):
                t = a_v[b, e, :] + b_v[b, e, :]
                t = jnp.where(t > 0.0, t, t * 0.2)
                pv = jnp.exp(t)
                p_v[b, e, :] = jnp.where(mask8, pv, 0.0)

            pltpu.sync_copy(p_v.at[b], pout.at[wid, pl.ds(jj * K, K)])
            pltpu.sync_copy(p_v.at[b], den.at[dst_v.at[jj]], add=True)

    plsc.subcore_barrier()
    pltpu.sync_copy(den.at[pl.ds(r0, RPT)], outd.at[c, pl.ds(r0, RPT)])


def _make_att1(h0):
    @functools.partial(
        pl.kernel,
        out_type=jax.ShapeDtypeStruct((NC, NP, DH), f32),
        mesh=_mesh(),
        compiler_params=_params,
        scratch_types=[
            pltpu.VMEM((CH, K), i32),
            pltpu.VMEM((CH, K), i32),
            pltpu.VMEM((2, K, DH), f32),
            pltpu.VMEM((2, K, 16), f32),
            pltpu.VMEM_SHARED((NP, DH), f32),
            pltpu.SemaphoreType.DMA,
            pltpu.SemaphoreType.DMA,
        ],
    )
    def att1(src3, dst3, zh, p1, zerosH, out,
             src_v, dst_v, rows_v, p_v, acc, sem, psem):
        c, s, wid, r0 = _wid_r0()
        pltpu.sync_copy(zerosH.at[pl.ds(r0, RPT)], acc.at[pl.ds(r0, RPT)])
        pltpu.sync_copy(src3.at[wid], src_v)
        pltpu.sync_copy(dst3.at[wid], dst_v)
        plsc.subcore_barrier()
        pltpu.async_copy(zh.at[src_v.at[0]], rows_v.at[0], sem)
        pltpu.async_copy(p1.at[wid, pl.ds(0, K)], p_v.at[0], psem)

        @pl.loop(0, CH, step=2)
        def _chunk(j):
            for b in range(2):
                jj = j + b

                @pl.when(jj + 1 < CH)
                def _pref():
                    pltpu.async_copy(zh.at[src_v.at[jj + 1]],
                                     rows_v.at[1 - b], sem)
                    pltpu.async_copy(p1.at[wid, pl.ds((jj + 1) * K, K)],
                                     p_v.at[1 - b], psem)

                pltpu.make_async_copy(zh.at[src_v.at[jj]],
                                      rows_v.at[b], sem).wait()
                pltpu.make_async_copy(p1.at[wid, pl.ds(jj * K, K)],
                                      p_v.at[b], psem).wait()

                @pl.loop(0, K)
                def _edge(e):
                    pv = p_v[b, e, :]
                    for h in range(4):
                        sp = _splat(pv, h0 + h)
                        sl = pl.ds(h * DH1, DH1)
                        rows_v[b, e, sl] = rows_v[b, e, sl] * sp

                pltpu.sync_copy(rows_v.at[b], acc.at[dst_v.at[jj]], add=True)

        plsc.subcore_barrier()
        pltpu.sync_copy(acc.at[pl.ds(r0, RPT)], out.at[c, pl.ds(r0, RPT)])

    return att1


_att1a = _make_att1(0)
_att1b = _make_att1(4)


@functools.partial(
    pl.kernel,
    out_type=(
        jax.ShapeDtypeStruct((NW, EWP, 16), f32),
        jax.ShapeDtypeStruct((NC, NP, 16), f32),
    ),
    mesh=_mesh(),
    compiler_params=_params,
    scratch_types=[
        pltpu.VMEM((CH, K), i32),
        pltpu.VMEM((CH, K), i32),
        pltpu.VMEM((NP,), f32),
        pltpu.VMEM((NP,), f32),
        pltpu.VMEM((K,), f32),
        pltpu.VMEM((K, 16), f32),
        pltpu.VMEM_SHARED((NP, 16), f32),
    ],
)
def _attp2_kernel(src3, dst3, es2, ed2, zeros16, pout, outd,
                  src_v, dst_v, es_v, ed_v, pg_v, p_v, den):
    c, s, wid, r0 = _wid_r0()
    pltpu.sync_copy(zeros16.at[pl.ds(r0, RPT)], den.at[pl.ds(r0, RPT)])
    pltpu.sync_copy(src3.at[wid], src_v)
    pltpu.sync_copy(dst3.at[wid], dst_v)
    pltpu.sync_copy(es2, es_v)
    pltpu.sync_copy(ed2, ed_v)
    plsc.subcore_barrier()

    @pl.loop(0, CH)
    def _chunk(j):
        @pl.loop(0, K // 16)
        def _group(m):
            srcs = src_v[j, pl.ds(m * 16, 16)]
            dsts = dst_v[j, pl.ds(m * 16, 16)]
            t = plsc.load_gather(es_v, [srcs]) + plsc.load_gather(ed_v, [dsts])
            t = jnp.where(t > 0.0, t, t * 0.2)
            pg_v[pl.ds(m * 16, 16)] = jnp.exp(t)

        @pl.loop(0, K)
        def _edge(e):
            p_v[e, :] = plsc.load_gather(pg_v, [jnp.full((16,), e, i32)])

        pltpu.sync_copy(p_v, pout.at[wid, pl.ds(j * K, K)])
        pltpu.sync_copy(p_v, den.at[dst_v.at[j]], add=True)

    plsc.subcore_barrier()
    pltpu.sync_copy(den.at[pl.ds(r0, RPT)], outd.at[c, pl.ds(r0, RPT)])


@functools.partial(
    pl.kernel,
    out_type=jax.ShapeDtypeStruct((NC, NP, D2), f32),
    mesh=_mesh(),
    compiler_params=_params,
    scratch_types=[
        pltpu.VMEM((CH, K), i32),
        pltpu.VMEM((CH, K), i32),
        pltpu.VMEM((2, K, D2), f32),
        pltpu.VMEM((2, K, 16), f32),
        pltpu.VMEM_SHARED((NP, D2), f32),
        pltpu.SemaphoreType.DMA,
        pltpu.SemaphoreType.DMA,
    ],
)
def _att2_kernel(src3, dst3, z2p, p2, zeros64, out,
                 src_v, dst_v, rows_v, p_v, acc, sem, psem):
    c, s, wid, r0 = _wid_r0()
    pltpu.sync_copy(zeros64.at[pl.ds(r0, RPT)], acc.at[pl.ds(r0, RPT)])
    pltpu.sync_copy(src3.at[wid], src_v)
    pltpu.sync_copy(dst3.at[wid], dst_v)
    plsc.subcore_barrier()
    pltpu.async_copy(z2p.at[src_v.at[0]], rows_v.at[0], sem)
    pltpu.async_copy(p2.at[wid, pl.ds(0, K)], p_v.at[0], psem)

    @pl.loop(0, CH, step=2)
    def _chunk(j):
        for b in range(2):
            jj = j + b

            @pl.when(jj + 1 < CH)
            def _pref():
                pltpu.async_copy(z2p.at[src_v.at[jj + 1]],
                                 rows_v.at[1 - b], sem)
                pltpu.async_copy(p2.at[wid, pl.ds((jj + 1) * K, K)],
                                 p_v.at[1 - b], psem)

            pltpu.make_async_copy(z2p.at[src_v.at[jj]],
                                  rows_v.at[b], sem).wait()
            pltpu.make_async_copy(p2.at[wid, pl.ds(jj * K, K)],
                                  p_v.at[b], psem).wait()

            @pl.loop(0, K)
            def _edge(e):
                sp = p_v[b, e, :]
                for q in range(D2 // 16):
                    sl = pl.ds(q * 16, 16)
                    rows_v[b, e, sl] = rows_v[b, e, sl] * sp

            pltpu.sync_copy(rows_v.at[b], acc.at[dst_v.at[jj]], add=True)

    plsc.subcore_barrier()
    pltpu.sync_copy(acc.at[pl.ds(r0, RPT)], out.at[c, pl.ds(r0, RPT)])


BLK = NP // 8
GRID = NP // BLK


def _rows(width):
    return pl.BlockSpec((BLK, width), lambda i: (i, 0))


def _prows(width):
    return pl.BlockSpec((2, BLK, width), lambda i: (0, i, 0))


def _full(shape):
    return pl.BlockSpec(shape, lambda i: tuple(0 for _ in shape))


def _inv_from_deg(dega_ref):
    deg = dega_ref[0, :, :1] + dega_ref[1, :, :1]
    base = pl.program_id(0) * BLK
    rowmask = base + lax.broadcasted_iota(i32, (BLK, 1), 0) < N
    return jnp.where(jnp.logical_and(rowmask, deg > 0.0),
                     lax.rsqrt(jnp.maximum(deg, 1.0)), 0.0)


def _t1_body(x_ref, br1_ref, s1_ref, dega_ref, za_ref, zb_ref):
    inv = _inv_from_deg(dega_ref)
    wc = jnp.dot(br1_ref[...], s1_ref[...], preferred_element_type=f32)
    z = jnp.dot(x_ref[...], wc, preferred_element_type=f32) * inv
    za_ref[...] = z[:, :DH]
    zb_ref[...] = z[:, DH:]


_t1 = pl.pallas_call(
    _t1_body,
    grid=(GRID,),
    in_specs=[_rows(NFEAT), _full((NFEAT, NBASE * DH1)),
              _full((NBASE * DH1, D1)), _prows(16)],
    out_specs=(_rows(DH), _rows(DH)),
    out_shape=(jax.ShapeDtypeStruct((NP, DH), f32),
               jax.ShapeDtypeStruct((NP, DH), f32)))


def _t2_body(acca_ref, accb_ref, dega_ref, asel_ref, bsel_ref,
             z1a_ref, z1b_ref, a_ref, b_ref):
    inv = _inv_from_deg(dega_ref)
    z1a = (acca_ref[0] + acca_ref[1]) * inv
    z1b = (accb_ref[0] + accb_ref[1]) * inv
    z1a_ref[...] = z1a
    z1b_ref[...] = z1b
    a_ref[...] = (jnp.dot(z1a, asel_ref[:DH, :], preferred_element_type=f32)
                  + jnp.dot(z1b, asel_ref[DH:, :],
                            preferred_element_type=f32))
    b_ref[...] = (jnp.dot(z1a, bsel_ref[:DH, :], preferred_element_type=f32)
                  + jnp.dot(z1b, bsel_ref[DH:, :],
                            preferred_element_type=f32))


_t2 = pl.pallas_call(
    _t2_body,
    grid=(GRID,),
    in_specs=[_prows(DH), _prows(DH), _prows(16),
              _full((D1, 16)), _full((D1, 16))],
    out_specs=(_rows(DH), _rows(DH), _rows(16), _rows(16)),
    out_shape=(jax.ShapeDtypeStruct((NP, DH), f32),
               jax.ShapeDtypeStruct((NP, DH), f32),
               jax.ShapeDtypeStruct((NP, 16), f32),
               jax.ShapeDtypeStruct((NP, 16), f32)))


def _t3_body(outa_ref, outb_ref, outd_ref, dega_ref, bias_ref, br2_ref,
             s2sel_ref, erep_ref, z2pre_ref):
    inv = _inv_from_deg(dega_ref)
    den = outd_ref[0] + outd_ref[1]
    div = jnp.dot(den[:, :H1], erep_ref[...], preferred_element_type=f32)
    num_a = outa_ref[0] + outa_ref[1]
    num_b = outb_ref[0] + outb_ref[1]
    ha = num_a / (div[:, :DH] + 1e-16) + bias_ref[:, :DH]
    hb = num_b / (div[:, DH:] + 1e-16) + bias_ref[:, DH:]
    ha = jnp.where(ha > 0.0, ha, jnp.exp(ha) - 1.0)
    hb = jnp.where(hb > 0.0, hb, jnp.exp(hb) - 1.0)
    wc2 = jnp.dot(br2_ref[...], s2sel_ref[...], preferred_element_type=f32)
    z2 = (jnp.dot(ha, wc2[:DH, :], preferred_element_type=f32)
          + jnp.dot(hb, wc2[DH:, :], preferred_element_type=f32))
    z2pre_ref[...] = z2 * inv


_t3 = pl.pallas_call(
    _t3_body,
    grid=(GRID,),
    in_specs=[_prows(DH), _prows(DH), _prows(16), _prows(16),
              _full((1, D1)), _full((D1, NBASE * D2)),
              _full((NBASE * D2, D2)), _full((H1, D1))],
    out_specs=_rows(D2),
    out_shape=jax.ShapeDtypeStruct((NP, D2), f32))


def _t4_body(acc_ref, dega_ref, a2c_ref, z2p_ref, s2_ref):
    inv = _inv_from_deg(dega_ref)
    z2p = (acc_ref[0] + acc_ref[1]) * inv
    z2p_ref[...] = z2p
    s2_ref[...] = jnp.dot(z2p, a2c_ref[...], preferred_element_type=f32)


_t4 = pl.pallas_call(
    _t4_body,
    grid=(GRID,),
    in_specs=[_prows(D2), _prows(16), _full((D2, 2))],
    out_specs=(_rows(D2), _rows(2)),
    out_shape=(jax.ShapeDtypeStruct((NP, D2), f32),
               jax.ShapeDtypeStruct((NP, 2), f32)))


def _t5_body(outp_ref, outd_ref, fin_ref):
    num = outp_ref[0] + outp_ref[1]
    den = outd_ref[0, :, :1] + outd_ref[1, :, :1]
    fin_ref[...] = num / (den + 1e-16)


_t5 = pl.pallas_call(
    _t5_body,
    grid=(GRID,),
    in_specs=[_prows(D2), _prows(16)],
    out_specs=_rows(D2),
    out_shape=jax.ShapeDtypeStruct((N, D2), f32))


def kernel(x, edge_index, basis1, coef1, a1_src, a1_dst, bias1,
           basis2, coef2, a2_src, a2_dst):
    src = edge_index[0].astype(i32)
    dst = edge_index[1].astype(i32)
    src3 = jnp.concatenate(
        [src.reshape(NW, EW), jnp.zeros((NW, EWP - EW), i32)],
        axis=1).reshape(NW, CH, K)
    dst3 = jnp.concatenate(
        [dst.reshape(NW, EW), jnp.full((NW, EWP - EW), N, i32)],
        axis=1).reshape(NW, CH, K)

    xp = jnp.pad(x, ((0, NP - N), (0, 0)))
    br1 = jnp.transpose(basis1, (1, 0, 2)).reshape(NFEAT, NBASE * DH1)
    s1 = jnp.kron(coef1.T, jnp.eye(DH1, dtype=f32))
    eyeh = jnp.eye(H1, dtype=f32)
    asel_l = (a1_src[:, :, None] * eyeh[:, None, :]).reshape(D1, H1)
    asel_r = (a1_dst[:, :, None] * eyeh[:, None, :]).reshape(D1, H1)
    ta_sel = jnp.concatenate([asel_l, asel_r], axis=1)
    tb_sel = jnp.concatenate([asel_r, asel_l], axis=1)
    br2 = jnp.transpose(basis2, (1, 0, 2)).reshape(D1, NBASE * D2)
    s2sel = jnp.kron(coef2.T, jnp.eye(D2, dtype=f32))
    erep = (eyeh[:, :, None] * jnp.ones((1, 1, DH1), f32)).reshape(H1, D1)
    a2cat = jnp.stack([a2_src[0], a2_dst[0]], axis=1)

    zeros64 = jnp.zeros((NP, DH), f32)
    zeros16 = jnp.zeros((NP, 16), f32)
    ones16 = jnp.ones((K, 16), f32)

    dega = _deg_kernel(dst3, zeros16, ones16)
    zpa, zpb = _t1(xp, br1, s1, dega)
    acca = _prop64(src3, dst3, zpa, zeros64)
    accb = _prop64(src3, dst3, zpb, zeros64)
    z1a, z1b, ta, tb = _t2(acca, accb, dega, ta_sel, tb_sel)
    p1, outd1 = _attp_kernel(src3, dst3, ta, tb, zeros16)
    outa = _att1a(src3, dst3, z1a, p1, zeros64)
    outb = _att1b(src3, dst3, z1b, p1, zeros64)
    z2pre = _t3(outa, outb, outd1, dega, bias1.reshape(1, D1), br2, s2sel,
                erep)
    acc2 = _prop64(src3, dst3, z2pre, zeros64)
    z2p, s2 = _t4(acc2, dega, a2cat)
    es2 = s2[:, 0]
    ed2 = s2[:, 1]
    p2, outd2 = _attp2_kernel(src3, dst3, es2, ed2, zeros16)
    out2 = _att2_kernel(src3, dst3, z2p, p2, zeros64)
    return _t5(out2, outd2)


# speedup vs baseline: 1.1984x; 1.0938x over previous
import functools

import jax
import jax.numpy as jnp
from jax import lax
from jax.experimental import pallas as pl
from jax.experimental.pallas import tpu as pltpu
from jax.experimental.pallas import tpu_sc as plsc

N = 10000
E = 320000
NFEAT = 128
H1 = 8
DH1 = 16
D1 = H1 * DH1
DH = 64
D2 = 64
NBASE = 4

NC = 2
NS = 16
NW = NC * NS
EW = E // NW
K = 128
CH = 80
EWP = CH * K
RPT = 632
NP = RPT * NS

f32 = jnp.float32
i32 = jnp.int32

_params = pltpu.CompilerParams(use_tc_tiling_on_sc=False,
                               needs_layout_passes=False)


def _mesh():
    return plsc.VectorSubcoreMesh(core_axis_name="c", subcore_axis_name="s")


_SPLAT_DNUMS = lax.GatherDimensionNumbers(
    offset_dims=(), collapsed_slice_dims=(0,), start_index_map=(0,))


def _splat(vec, h):
    idx = jnp.full((16, 1), h, dtype=i32)
    return lax.gather(vec, idx, _SPLAT_DNUMS, (1,),
                      mode=lax.GatherScatterMode.PROMISE_IN_BOUNDS)


def _wid_r0():
    c = lax.axis_index("c")
    s = lax.axis_index("s")
    return c, s, c * NS + s, s * RPT


@functools.partial(
    pl.kernel,
    out_type=jax.ShapeDtypeStruct((NC, NP, 16), f32),
    mesh=_mesh(),
    compiler_params=_params,
    scratch_types=[
        pltpu.VMEM((CH, K), i32),
        pltpu.VMEM((K, 16), f32),
        pltpu.VMEM_SHARED((NP, 16), f32),
    ],
)
def _deg_kernel(dst3, zeros16, ones16, out, dst_v, ones_v, acc):
    c, s, wid, r0 = _wid_r0()
    pltpu.sync_copy(zeros16.at[pl.ds(r0, RPT)], acc.at[pl.ds(r0, RPT)])
    pltpu.sync_copy(dst3.at[wid], dst_v)
    pltpu.sync_copy(ones16, ones_v)
    plsc.subcore_barrier()

    @pl.loop(0, CH)
    def _chunk(j):
        pltpu.sync_copy(ones_v, acc.at[dst_v.at[j]], add=True)

    plsc.subcore_barrier()
    pltpu.sync_copy(acc.at[pl.ds(r0, RPT)], out.at[c, pl.ds(r0, RPT)])


def _make_prop(D):
    @functools.partial(
        pl.kernel,
        out_type=jax.ShapeDtypeStruct((NC, NP, D), f32),
        mesh=_mesh(),
        compiler_params=_params,
        scratch_types=[
            pltpu.VMEM((CH, K), i32),
            pltpu.VMEM((CH, K), i32),
            pltpu.VMEM((2, K, D), f32),
            pltpu.VMEM_SHARED((NP, D), f32),
            pltpu.SemaphoreType.DMA,
        ],
    )
    def prop(src3, dst3, z, zerosD, out, src_v, dst_v, rows_v, acc, sem):
        c, s, wid, r0 = _wid_r0()
        pltpu.sync_copy(zerosD.at[pl.ds(r0, RPT)], acc.at[pl.ds(r0, RPT)])
        pltpu.sync_copy(src3.at[wid], src_v)
        pltpu.sync_copy(dst3.at[wid], dst_v)
        plsc.subcore_barrier()
        pltpu.async_copy(z.at[src_v.at[0]], rows_v.at[0], sem)

        @pl.loop(0, CH, step=2)
        def _chunk(j):
            for b in range(2):
                jj = j + b

                @pl.when(jj + 1 < CH)
                def _pref():
                    pltpu.async_copy(z.at[src_v.at[jj + 1]],
                                     rows_v.at[1 - b], sem)

                pltpu.make_async_copy(z.at[src_v.at[jj]],
                                      rows_v.at[b], sem).wait()
                pltpu.sync_copy(rows_v.at[b], acc.at[dst_v.at[jj]], add=True)

        plsc.subcore_barrier()
        pltpu.sync_copy(acc.at[pl.ds(r0, RPT)], out.at[c, pl.ds(r0, RPT)])

    return prop


_prop64 = _make_prop(DH)


@functools.partial(
    pl.kernel,
    out_type=(
        jax.ShapeDtypeStruct((NW, EWP, 16), f32),
        jax.ShapeDtypeStruct((NC, NP, 16), f32),
    ),
    mesh=_mesh(),
    compiler_params=_params,
    scratch_types=[
        pltpu.VMEM((CH, K), i32),
        pltpu.VMEM((CH, K), i32),
        pltpu.VMEM((2, K, 16), f32),
        pltpu.VMEM((2, K, 16), f32),
        pltpu.VMEM((2, K, 16), f32),
        pltpu.VMEM_SHARED((NP, 16), f32),
        pltpu.SemaphoreType.DMA,
    ],
)
def _attp_kernel(src3, dst3, ta, tb, zeros16, pout, outd,
                 src_v, dst_v, a_v, b_v, p_v, den, sem):
    c, s, wid, r0 = _wid_r0()
    pltpu.sync_copy(zeros16.at[pl.ds(r0, RPT)], den.at[pl.ds(r0, RPT)])
    pltpu.sync_copy(src3.at[wid], src_v)
    pltpu.sync_copy(dst3.at[wid], dst_v)
    plsc.subcore_barrier()
    lane = lax.iota(i32, 16)
    mask8 = lane < 8
    pltpu.async_copy(ta.at[src_v.at[0]], a_v.at[0], sem)
    pltpu.async_copy(tb.at[dst_v.at[0]], b_v.at[0], sem)

    @pl.loop(0, CH, step=2)
    def _chunk(j):
        for b in range(2):
            jj = j + b

            @pl.when(jj + 1 < CH)
            def _pref():
                pltpu.async_copy(ta.at[src_v.at[jj + 1]], a_v.at[1 - b], sem)
                pltpu.async_copy(tb.at[dst_v.at[jj + 1]], b_v.at[1 - b], sem)

            pltpu.make_async_copy(ta.at[src_v.at[jj]], a_v.at[b], sem).wait()
            pltpu.make_async_copy(tb.at[dst_v.at[jj]], b_v.at[b], sem).wait()

            @plsc.parallel_loop(0, K, unroll=4)
            def _edge(e):
                t = a_v[b, e, :] + b_v[b, e, :]
                t = jnp.where(t > 0.0, t, t * 0.2)
                pv = jnp.exp(t)
                p_v[b, e, :] = jnp.where(mask8, pv, 0.0)

            pltpu.sync_copy(p_v.at[b], pout.at[wid, pl.ds(jj * K, K)])
            pltpu.sync_copy(p_v.at[b], den.at[dst_v.at[jj]], add=True)

    plsc.subcore_barrier()
    pltpu.sync_copy(den.at[pl.ds(r0, RPT)], outd.at[c, pl.ds(r0, RPT)])


def _make_att1(h0):
    @functools.partial(
        pl.kernel,
        out_type=jax.ShapeDtypeStruct((NC, NP, DH), f32),
        mesh=_mesh(),
        compiler_params=_params,
        scratch_types=[
            pltpu.VMEM((CH, K), i32),
            pltpu.VMEM((CH, K), i32),
            pltpu.VMEM((2, K, DH), f32),
            pltpu.VMEM((2, K, 16), f32),
            pltpu.VMEM_SHARED((NP, DH), f32),
            pltpu.SemaphoreType.DMA,
            pltpu.SemaphoreType.DMA,
        ],
    )
    def att1(src3, dst3, zh, p1, zerosH, out,
             src_v, dst_v, rows_v, p_v, acc, sem, psem):
        c, s, wid, r0 = _wid_r0()
        pltpu.sync_copy(zerosH.at[pl.ds(r0, RPT)], acc.at[pl.ds(r0, RPT)])
        pltpu.sync_copy(src3.at[wid], src_v)
        pltpu.sync_copy(dst3.at[wid], dst_v)
        plsc.subcore_barrier()
        pltpu.async_copy(zh.at[src_v.at[0]], rows_v.at[0], sem)
        pltpu.async_copy(p1.at[wid, pl.ds(0, K)], p_v.at[0], psem)

        @pl.loop(0, CH, step=2)
        def _chunk(j):
            for b in range(2):
                jj = j + b

                @pl.when(jj + 1 < CH)
                def _pref():
                    pltpu.async_copy(zh.at[src_v.at[jj + 1]],
                                     rows_v.at[1 - b], sem)
                    pltpu.async_copy(p1.at[wid, pl.ds((jj + 1) * K, K)],
                                     p_v.at[1 - b], psem)

                pltpu.make_async_copy(zh.at[src_v.at[jj]],
                                      rows_v.at[b], sem).wait()
                pltpu.make_async_copy(p1.at[wid, pl.ds(jj * K, K)],
                                      p_v.at[b], psem).wait()

                @plsc.parallel_loop(0, K, unroll=4)
                def _edge(e):
                    pv = p_v[b, e, :]
                    for h in range(4):
                        sp = _splat(pv, h0 + h)
                        sl = pl.ds(h * DH1, DH1)
                        rows_v[b, e, sl] = rows_v[b, e, sl] * sp

                pltpu.sync_copy(rows_v.at[b], acc.at[dst_v.at[jj]], add=True)

        plsc.subcore_barrier()
        pltpu.sync_copy(acc.at[pl.ds(r0, RPT)], out.at[c, pl.ds(r0, RPT)])

    return att1


_att1a = _make_att1(0)
_att1b = _make_att1(4)


@functools.partial(
    pl.kernel,
    out_type=(
        jax.ShapeDtypeStruct((NW, EWP, 16), f32),
        jax.ShapeDtypeStruct((NC, NP, 16), f32),
    ),
    mesh=_mesh(),
    compiler_params=_params,
    scratch_types=[
        pltpu.VMEM((CH, K), i32),
        pltpu.VMEM((CH, K), i32),
        pltpu.VMEM((NP,), f32),
        pltpu.VMEM((NP,), f32),
        pltpu.VMEM((K,), f32),
        pltpu.VMEM((K, 16), f32),
        pltpu.VMEM_SHARED((NP, 16), f32),
    ],
)
def _attp2_kernel(src3, dst3, es2, ed2, zeros16, pout, outd,
                  src_v, dst_v, es_v, ed_v, pg_v, p_v, den):
    c, s, wid, r0 = _wid_r0()
    pltpu.sync_copy(zeros16.at[pl.ds(r0, RPT)], den.at[pl.ds(r0, RPT)])
    pltpu.sync_copy(src3.at[wid], src_v)
    pltpu.sync_copy(dst3.at[wid], dst_v)
    pltpu.sync_copy(es2, es_v)
    pltpu.sync_copy(ed2, ed_v)
    plsc.subcore_barrier()

    @pl.loop(0, CH)
    def _chunk(j):
        @plsc.parallel_loop(0, K // 16, unroll=4)
        def _group(m):
            srcs = src_v[j, pl.ds(m * 16, 16)]
            dsts = dst_v[j, pl.ds(m * 16, 16)]
            t = plsc.load_gather(es_v, [srcs]) + plsc.load_gather(ed_v, [dsts])
            t = jnp.where(t > 0.0, t, t * 0.2)
            pg_v[pl.ds(m * 16, 16)] = jnp.exp(t)

        @plsc.parallel_loop(0, K, unroll=4)
        def _edge(e):
            p_v[e, :] = plsc.load_gather(pg_v, [jnp.full((16,), e, i32)])

        pltpu.sync_copy(p_v, pout.at[wid, pl.ds(j * K, K)])
        pltpu.sync_copy(p_v, den.at[dst_v.at[j]], add=True)

    plsc.subcore_barrier()
    pltpu.sync_copy(den.at[pl.ds(r0, RPT)], outd.at[c, pl.ds(r0, RPT)])


@functools.partial(
    pl.kernel,
    out_type=jax.ShapeDtypeStruct((NC, NP, D2), f32),
    mesh=_mesh(),
    compiler_params=_params,
    scratch_types=[
        pltpu.VMEM((CH, K), i32),
        pltpu.VMEM((CH, K), i32),
        pltpu.VMEM((2, K, D2), f32),
        pltpu.VMEM((2, K, 16), f32),
        pltpu.VMEM_SHARED((NP, D2), f32),
        pltpu.SemaphoreType.DMA,
        pltpu.SemaphoreType.DMA,
    ],
)
def _att2_kernel(src3, dst3, z2p, p2, zeros64, out,
                 src_v, dst_v, rows_v, p_v, acc, sem, psem):
    c, s, wid, r0 = _wid_r0()
    pltpu.sync_copy(zeros64.at[pl.ds(r0, RPT)], acc.at[pl.ds(r0, RPT)])
    pltpu.sync_copy(src3.at[wid], src_v)
    pltpu.sync_copy(dst3.at[wid], dst_v)
    plsc.subcore_barrier()
    pltpu.async_copy(z2p.at[src_v.at[0]], rows_v.at[0], sem)
    pltpu.async_copy(p2.at[wid, pl.ds(0, K)], p_v.at[0], psem)

    @pl.loop(0, CH, step=2)
    def _chunk(j):
        for b in range(2):
            jj = j + b

            @pl.when(jj + 1 < CH)
            def _pref():
                pltpu.async_copy(z2p.at[src_v.at[jj + 1]],
                                 rows_v.at[1 - b], sem)
                pltpu.async_copy(p2.at[wid, pl.ds((jj + 1) * K, K)],
                                 p_v.at[1 - b], psem)

            pltpu.make_async_copy(z2p.at[src_v.at[jj]],
                                  rows_v.at[b], sem).wait()
            pltpu.make_async_copy(p2.at[wid, pl.ds(jj * K, K)],
                                  p_v.at[b], psem).wait()

            @plsc.parallel_loop(0, K, unroll=4)
            def _edge(e):
                sp = p_v[b, e, :]
                for q in range(D2 // 16):
                    sl = pl.ds(q * 16, 16)
                    rows_v[b, e, sl] = rows_v[b, e, sl] * sp

            pltpu.sync_copy(rows_v.at[b], acc.at[dst_v.at[jj]], add=True)

    plsc.subcore_barrier()
    pltpu.sync_copy(acc.at[pl.ds(r0, RPT)], out.at[c, pl.ds(r0, RPT)])


BLK = NP // 8
GRID = NP // BLK


def _rows(width):
    return pl.BlockSpec((BLK, width), lambda i: (i, 0))


def _prows(width):
    return pl.BlockSpec((2, BLK, width), lambda i: (0, i, 0))


def _full(shape):
    return pl.BlockSpec(shape, lambda i: tuple(0 for _ in shape))


def _inv_from_deg(dega_ref):
    deg = dega_ref[0, :, :1] + dega_ref[1, :, :1]
    base = pl.program_id(0) * BLK
    rowmask = base + lax.broadcasted_iota(i32, (BLK, 1), 0) < N
    return jnp.where(jnp.logical_and(rowmask, deg > 0.0),
                     lax.rsqrt(jnp.maximum(deg, 1.0)), 0.0)


def _t1_body(x_ref, br1_ref, s1_ref, dega_ref, za_ref, zb_ref):
    inv = _inv_from_deg(dega_ref)
    wc = jnp.dot(br1_ref[...], s1_ref[...], preferred_element_type=f32)
    z = jnp.dot(x_ref[...], wc, preferred_element_type=f32) * inv
    za_ref[...] = z[:, :DH]
    zb_ref[...] = z[:, DH:]


_t1 = pl.pallas_call(
    _t1_body,
    grid=(GRID,),
    in_specs=[_rows(NFEAT), _full((NFEAT, NBASE * DH1)),
              _full((NBASE * DH1, D1)), _prows(16)],
    out_specs=(_rows(DH), _rows(DH)),
    out_shape=(jax.ShapeDtypeStruct((NP, DH), f32),
               jax.ShapeDtypeStruct((NP, DH), f32)))


def _t2_body(acca_ref, accb_ref, dega_ref, asel_ref, bsel_ref,
             z1a_ref, z1b_ref, a_ref, b_ref):
    inv = _inv_from_deg(dega_ref)
    z1a = (acca_ref[0] + acca_ref[1]) * inv
    z1b = (accb_ref[0] + accb_ref[1]) * inv
    z1a_ref[...] = z1a
    z1b_ref[...] = z1b
    a_ref[...] = (jnp.dot(z1a, asel_ref[:DH, :], preferred_element_type=f32)
                  + jnp.dot(z1b, asel_ref[DH:, :],
                            preferred_element_type=f32))
    b_ref[...] = (jnp.dot(z1a, bsel_ref[:DH, :], preferred_element_type=f32)
                  + jnp.dot(z1b, bsel_ref[DH:, :],
                            preferred_element_type=f32))


_t2 = pl.pallas_call(
    _t2_body,
    grid=(GRID,),
    in_specs=[_prows(DH), _prows(DH), _prows(16),
              _full((D1, 16)), _full((D1, 16))],
    out_specs=(_rows(DH), _rows(DH), _rows(16), _rows(16)),
    out_shape=(jax.ShapeDtypeStruct((NP, DH), f32),
               jax.ShapeDtypeStruct((NP, DH), f32),
               jax.ShapeDtypeStruct((NP, 16), f32),
               jax.ShapeDtypeStruct((NP, 16), f32)))


def _t3_body(outa_ref, outb_ref, outd_ref, dega_ref, bias_ref, br2_ref,
             s2sel_ref, erep_ref, z2pre_ref):
    inv = _inv_from_deg(dega_ref)
    den = outd_ref[0] + outd_ref[1]
    div = jnp.dot(den[:, :H1], erep_ref[...], preferred_element_type=f32)
    num_a = outa_ref[0] + outa_ref[1]
    num_b = outb_ref[0] + outb_ref[1]
    ha = num_a / (div[:, :DH] + 1e-16) + bias_ref[:, :DH]
    hb = num_b / (div[:, DH:] + 1e-16) + bias_ref[:, DH:]
    ha = jnp.where(ha > 0.0, ha, jnp.exp(ha) - 1.0)
    hb = jnp.where(hb > 0.0, hb, jnp.exp(hb) - 1.0)
    wc2 = jnp.dot(br2_ref[...], s2sel_ref[...], preferred_element_type=f32)
    z2 = (jnp.dot(ha, wc2[:DH, :], preferred_element_type=f32)
          + jnp.dot(hb, wc2[DH:, :], preferred_element_type=f32))
    z2pre_ref[...] = z2 * inv


_t3 = pl.pallas_call(
    _t3_body,
    grid=(GRID,),
    in_specs=[_prows(DH), _prows(DH), _prows(16), _prows(16),
              _full((1, D1)), _full((D1, NBASE * D2)),
              _full((NBASE * D2, D2)), _full((H1, D1))],
    out_specs=_rows(D2),
    out_shape=jax.ShapeDtypeStruct((NP, D2), f32))


def _t4_body(acc_ref, dega_ref, a2c_ref, z2p_ref, s2_ref):
    inv = _inv_from_deg(dega_ref)
    z2p = (acc_ref[0] + acc_ref[1]) * inv
    z2p_ref[...] = z2p
    s2_ref[...] = jnp.dot(z2p, a2c_ref[...], preferred_element_type=f32)


_t4 = pl.pallas_call(
    _t4_body,
    grid=(GRID,),
    in_specs=[_prows(D2), _prows(16), _full((D2, 2))],
    out_specs=(_rows(D2), _rows(2)),
    out_shape=(jax.ShapeDtypeStruct((NP, D2), f32),
               jax.ShapeDtypeStruct((NP, 2), f32)))


def _t5_body(outp_ref, outd_ref, fin_ref):
    num = outp_ref[0] + outp_ref[1]
    den = outd_ref[0, :, :1] + outd_ref[1, :, :1]
    fin_ref[...] = num / (den + 1e-16)


_t5 = pl.pallas_call(
    _t5_body,
    grid=(GRID,),
    in_specs=[_prows(D2), _prows(16)],
    out_specs=_rows(D2),
    out_shape=jax.ShapeDtypeStruct((N, D2), f32))


def kernel(x, edge_index, basis1, coef1, a1_src, a1_dst, bias1,
           basis2, coef2, a2_src, a2_dst):
    src = edge_index[0].astype(i32)
    dst = edge_index[1].astype(i32)
    src3 = jnp.concatenate(
        [src.reshape(NW, EW), jnp.zeros((NW, EWP - EW), i32)],
        axis=1).reshape(NW, CH, K)
    dst3 = jnp.concatenate(
        [dst.reshape(NW, EW), jnp.full((NW, EWP - EW), N, i32)],
        axis=1).reshape(NW, CH, K)

    xp = jnp.pad(x, ((0, NP - N), (0, 0)))
    br1 = jnp.transpose(basis1, (1, 0, 2)).reshape(NFEAT, NBASE * DH1)
    s1 = jnp.kron(coef1.T, jnp.eye(DH1, dtype=f32))
    eyeh = jnp.eye(H1, dtype=f32)
    asel_l = (a1_src[:, :, None] * eyeh[:, None, :]).reshape(D1, H1)
    asel_r = (a1_dst[:, :, None] * eyeh[:, None, :]).reshape(D1, H1)
    ta_sel = jnp.concatenate([asel_l, asel_r], axis=1)
    tb_sel = jnp.concatenate([asel_r, asel_l], axis=1)
    br2 = jnp.transpose(basis2, (1, 0, 2)).reshape(D1, NBASE * D2)
    s2sel = jnp.kron(coef2.T, jnp.eye(D2, dtype=f32))
    erep = (eyeh[:, :, None] * jnp.ones((1, 1, DH1), f32)).reshape(H1, D1)
    a2cat = jnp.stack([a2_src[0], a2_dst[0]], axis=1)

    zeros64 = jnp.zeros((NP, DH), f32)
    zeros16 = jnp.zeros((NP, 16), f32)
    ones16 = jnp.ones((K, 16), f32)

    dega = _deg_kernel(dst3, zeros16, ones16)
    zpa, zpb = _t1(xp, br1, s1, dega)
    acca = _prop64(src3, dst3, zpa, zeros64)
    accb = _prop64(src3, dst3, zpb, zeros64)
    z1a, z1b, ta, tb = _t2(acca, accb, dega, ta_sel, tb_sel)
    p1, outd1 = _attp_kernel(src3, dst3, ta, tb, zeros16)
    outa = _att1a(src3, dst3, z1a, p1, zeros64)
    outb = _att1b(src3, dst3, z1b, p1, zeros64)
    z2pre = _t3(outa, outb, outd1, dega, bias1.reshape(1, D1), br2, s2sel,
                erep)
    acc2 = _prop64(src3, dst3, z2pre, zeros64)
    z2p, s2 = _t4(acc2, dega, a2cat)
    es2 = s2[:, 0]
    ed2 = s2[:, 1]
    p2, outd2 = _attp2_kernel(src3, dst3, es2, ed2, zeros16)
    out2 = _att2_kernel(src3, dst3, z2p, p2, zeros64)
    return _t5(out2, outd2)


# speedup vs baseline: 1.2755x; 1.0643x over previous
import functools

import jax
import jax.numpy as jnp
from jax import lax
from jax.experimental import pallas as pl
from jax.experimental.pallas import tpu as pltpu
from jax.experimental.pallas import tpu_sc as plsc

N = 10000
E = 320000
NFEAT = 128
H1 = 8
DH1 = 16
D1 = H1 * DH1
DH = 64
D2 = 64
NBASE = 4

NC = 2
NS = 16
NW = NC * NS
EW = E // NW
K = 128
CH = 80
EWP = CH * K
RPT = 632
NP = RPT * NS

f32 = jnp.float32
i32 = jnp.int32

_params = pltpu.CompilerParams(use_tc_tiling_on_sc=False,
                               needs_layout_passes=False)


def _mesh():
    return plsc.VectorSubcoreMesh(core_axis_name="c", subcore_axis_name="s")


_SPLAT_DNUMS = lax.GatherDimensionNumbers(
    offset_dims=(), collapsed_slice_dims=(0,), start_index_map=(0,))


def _splat(vec, h):
    idx = jnp.full((16, 1), h, dtype=i32)
    return lax.gather(vec, idx, _SPLAT_DNUMS, (1,),
                      mode=lax.GatherScatterMode.PROMISE_IN_BOUNDS)


def _wid_r0():
    c = lax.axis_index("c")
    s = lax.axis_index("s")
    return c, s, c * NS + s, s * RPT


@functools.partial(
    pl.kernel,
    out_type=jax.ShapeDtypeStruct((NC, NP, 16), f32),
    mesh=_mesh(),
    compiler_params=_params,
    scratch_types=[
        pltpu.VMEM((CH, K), i32),
        pltpu.VMEM((K, 16), f32),
        pltpu.VMEM_SHARED((NP, 16), f32),
        pltpu.SemaphoreType.DMA,
    ],
)
def _deg_kernel(dst3, zeros16, ones16, out, dst_v, ones_v, acc, dsem):
    c, s, wid, r0 = _wid_r0()
    pltpu.sync_copy(zeros16.at[pl.ds(r0, RPT)], acc.at[pl.ds(r0, RPT)])
    pltpu.sync_copy(dst3.at[wid], dst_v)
    pltpu.sync_copy(ones16, ones_v)
    plsc.subcore_barrier()

    @pl.loop(0, CH, step=8)
    def _chunk(j):
        for b in range(8):
            pltpu.async_copy(ones_v, acc.at[dst_v.at[j + b]], dsem, add=True)
        for b in range(8):
            pltpu.make_async_copy(ones_v, acc.at[dst_v.at[j]], dsem).wait()

    plsc.subcore_barrier()
    pltpu.sync_copy(acc.at[pl.ds(r0, RPT)], out.at[c, pl.ds(r0, RPT)])


def _make_prop(D):
    @functools.partial(
        pl.kernel,
        out_type=jax.ShapeDtypeStruct((NC, NP, D), f32),
        mesh=_mesh(),
        compiler_params=_params,
        scratch_types=[
            pltpu.VMEM((CH, K), i32),
            pltpu.VMEM((CH, K), i32),
            pltpu.VMEM((4, K, D), f32),
            pltpu.VMEM_SHARED((NP, D), f32),
            pltpu.SemaphoreType.DMA,
        ],
    )
    def prop(src3, dst3, z, zerosD, out, src_v, dst_v, rows_v, acc, sem):
        c, s, wid, r0 = _wid_r0()
        pltpu.sync_copy(zerosD.at[pl.ds(r0, RPT)], acc.at[pl.ds(r0, RPT)])
        pltpu.sync_copy(src3.at[wid], src_v)
        pltpu.sync_copy(dst3.at[wid], dst_v)
        plsc.subcore_barrier()
        for q in range(3):
            pltpu.async_copy(z.at[src_v.at[q]], rows_v.at[q], sem)

        @pl.loop(0, CH, step=4)
        def _chunk(j):
            for b in range(4):
                jj = j + b

                @pl.when(jj + 3 < CH)
                def _pref():
                    pltpu.async_copy(z.at[src_v.at[jj + 3]],
                                     rows_v.at[(b + 3) % 4], sem)

                pltpu.make_async_copy(z.at[src_v.at[jj]],
                                      rows_v.at[b], sem).wait()
                pltpu.sync_copy(rows_v.at[b], acc.at[dst_v.at[jj]], add=True)

        plsc.subcore_barrier()
        pltpu.sync_copy(acc.at[pl.ds(r0, RPT)], out.at[c, pl.ds(r0, RPT)])

    return prop


_prop64 = _make_prop(DH)


@functools.partial(
    pl.kernel,
    out_type=(
        jax.ShapeDtypeStruct((NW, EWP, 16), f32),
        jax.ShapeDtypeStruct((NC, NP, 16), f32),
    ),
    mesh=_mesh(),
    compiler_params=_params,
    scratch_types=[
        pltpu.VMEM((CH, K), i32),
        pltpu.VMEM((CH, K), i32),
        pltpu.VMEM((4, K, 16), f32),
        pltpu.VMEM((4, K, 16), f32),
        pltpu.VMEM((4, K, 16), f32),
        pltpu.VMEM_SHARED((NP, 16), f32),
        pltpu.SemaphoreType.DMA,
    ],
)
def _attp_kernel(src3, dst3, ta, tb, zeros16, pout, outd,
                 src_v, dst_v, a_v, b_v, p_v, den, sem):
    c, s, wid, r0 = _wid_r0()
    pltpu.sync_copy(zeros16.at[pl.ds(r0, RPT)], den.at[pl.ds(r0, RPT)])
    pltpu.sync_copy(src3.at[wid], src_v)
    pltpu.sync_copy(dst3.at[wid], dst_v)
    plsc.subcore_barrier()
    lane = lax.iota(i32, 16)
    mask8 = lane < 8
    for q in range(3):
        pltpu.async_copy(ta.at[src_v.at[q]], a_v.at[q], sem)
        pltpu.async_copy(tb.at[dst_v.at[q]], b_v.at[q], sem)

    @pl.loop(0, CH, step=4)
    def _chunk(j):
        for b in range(4):
            jj = j + b

            @pl.when(jj + 3 < CH)
            def _pref():
                pltpu.async_copy(ta.at[src_v.at[jj + 3]],
                                 a_v.at[(b + 3) % 4], sem)
                pltpu.async_copy(tb.at[dst_v.at[jj + 3]],
                                 b_v.at[(b + 3) % 4], sem)

            pltpu.make_async_copy(ta.at[src_v.at[jj]], a_v.at[b], sem).wait()
            pltpu.make_async_copy(tb.at[dst_v.at[jj]], b_v.at[b], sem).wait()

            @plsc.parallel_loop(0, K, unroll=4)
            def _edge(e):
                t = a_v[b, e, :] + b_v[b, e, :]
                t = jnp.where(t > 0.0, t, t * 0.2)
                pv = jnp.exp(t)
                p_v[b, e, :] = jnp.where(mask8, pv, 0.0)

            pltpu.sync_copy(p_v.at[b], pout.at[wid, pl.ds(jj * K, K)])
            pltpu.sync_copy(p_v.at[b], den.at[dst_v.at[jj]], add=True)

    plsc.subcore_barrier()
    pltpu.sync_copy(den.at[pl.ds(r0, RPT)], outd.at[c, pl.ds(r0, RPT)])


def _make_att1(h0):
    @functools.partial(
        pl.kernel,
        out_type=jax.ShapeDtypeStruct((NC, NP, DH), f32),
        mesh=_mesh(),
        compiler_params=_params,
        scratch_types=[
            pltpu.VMEM((CH, K), i32),
            pltpu.VMEM((CH, K), i32),
            pltpu.VMEM((4, K, DH), f32),
            pltpu.VMEM((4, K, 16), f32),
            pltpu.VMEM_SHARED((NP, DH), f32),
            pltpu.SemaphoreType.DMA,
            pltpu.SemaphoreType.DMA,
        ],
    )
    def att1(src3, dst3, zh, p1, zerosH, out,
             src_v, dst_v, rows_v, p_v, acc, sem, psem):
        c, s, wid, r0 = _wid_r0()
        pltpu.sync_copy(zerosH.at[pl.ds(r0, RPT)], acc.at[pl.ds(r0, RPT)])
        pltpu.sync_copy(src3.at[wid], src_v)
        pltpu.sync_copy(dst3.at[wid], dst_v)
        plsc.subcore_barrier()
        for q in range(3):
            pltpu.async_copy(zh.at[src_v.at[q]], rows_v.at[q], sem)
            pltpu.async_copy(p1.at[wid, pl.ds(q * K, K)], p_v.at[q], psem)

        @pl.loop(0, CH, step=4)
        def _chunk(j):
            for b in range(4):
                jj = j + b

                @pl.when(jj + 3 < CH)
                def _pref():
                    pltpu.async_copy(zh.at[src_v.at[jj + 3]],
                                     rows_v.at[(b + 3) % 4], sem)
                    pltpu.async_copy(p1.at[wid, pl.ds((jj + 3) * K, K)],
                                     p_v.at[(b + 3) % 4], psem)

                pltpu.make_async_copy(zh.at[src_v.at[jj]],
                                      rows_v.at[b], sem).wait()
                pltpu.make_async_copy(p1.at[wid, pl.ds(jj * K, K)],
                                      p_v.at[b], psem).wait()

                @plsc.parallel_loop(0, K, unroll=4)
                def _edge(e):
                    pv = p_v[b, e, :]
                    for h in range(4):
                        sp = _splat(pv, h0 + h)
                        sl = pl.ds(h * DH1, DH1)
                        rows_v[b, e, sl] = rows_v[b, e, sl] * sp

                pltpu.sync_copy(rows_v.at[b], acc.at[dst_v.at[jj]], add=True)

        plsc.subcore_barrier()
        pltpu.sync_copy(acc.at[pl.ds(r0, RPT)], out.at[c, pl.ds(r0, RPT)])

    return att1


_att1a = _make_att1(0)
_att1b = _make_att1(4)


@functools.partial(
    pl.kernel,
    out_type=(
        jax.ShapeDtypeStruct((NW, EWP, 16), f32),
        jax.ShapeDtypeStruct((NC, NP, 16), f32),
    ),
    mesh=_mesh(),
    compiler_params=_params,
    scratch_types=[
        pltpu.VMEM((CH, K), i32),
        pltpu.VMEM((CH, K), i32),
        pltpu.VMEM((NP,), f32),
        pltpu.VMEM((NP,), f32),
        pltpu.VMEM((K,), f32),
        pltpu.VMEM((K, 16), f32),
        pltpu.VMEM_SHARED((NP, 16), f32),
    ],
)
def _attp2_kernel(src3, dst3, es2, ed2, zeros16, pout, outd,
                  src_v, dst_v, es_v, ed_v, pg_v, p_v, den):
    c, s, wid, r0 = _wid_r0()
    pltpu.sync_copy(zeros16.at[pl.ds(r0, RPT)], den.at[pl.ds(r0, RPT)])
    pltpu.sync_copy(src3.at[wid], src_v)
    pltpu.sync_copy(dst3.at[wid], dst_v)
    pltpu.sync_copy(es2, es_v)
    pltpu.sync_copy(ed2, ed_v)
    plsc.subcore_barrier()

    @pl.loop(0, CH)
    def _chunk(j):
        @plsc.parallel_loop(0, K // 16, unroll=4)
        def _group(m):
            srcs = src_v[j, pl.ds(m * 16, 16)]
            dsts = dst_v[j, pl.ds(m * 16, 16)]
            t = plsc.load_gather(es_v, [srcs]) + plsc.load_gather(ed_v, [dsts])
            t = jnp.where(t > 0.0, t, t * 0.2)
            pg_v[pl.ds(m * 16, 16)] = jnp.exp(t)

        @plsc.parallel_loop(0, K, unroll=4)
        def _edge(e):
            p_v[e, :] = plsc.load_gather(pg_v, [jnp.full((16,), e, i32)])

        pltpu.sync_copy(p_v, pout.at[wid, pl.ds(j * K, K)])
        pltpu.sync_copy(p_v, den.at[dst_v.at[j]], add=True)

    plsc.subcore_barrier()
    pltpu.sync_copy(den.at[pl.ds(r0, RPT)], outd.at[c, pl.ds(r0, RPT)])


@functools.partial(
    pl.kernel,
    out_type=jax.ShapeDtypeStruct((NC, NP, D2), f32),
    mesh=_mesh(),
    compiler_params=_params,
    scratch_types=[
        pltpu.VMEM((CH, K), i32),
        pltpu.VMEM((CH, K), i32),
        pltpu.VMEM((4, K, D2), f32),
        pltpu.VMEM((4, K, 16), f32),
        pltpu.VMEM_SHARED((NP, D2), f32),
        pltpu.SemaphoreType.DMA,
        pltpu.SemaphoreType.DMA,
    ],
)
def _att2_kernel(src3, dst3, z2p, p2, zeros64, out,
                 src_v, dst_v, rows_v, p_v, acc, sem, psem):
    c, s, wid, r0 = _wid_r0()
    pltpu.sync_copy(zeros64.at[pl.ds(r0, RPT)], acc.at[pl.ds(r0, RPT)])
    pltpu.sync_copy(src3.at[wid], src_v)
    pltpu.sync_copy(dst3.at[wid], dst_v)
    plsc.subcore_barrier()
    for q in range(3):
        pltpu.async_copy(z2p.at[src_v.at[q]], rows_v.at[q], sem)
        pltpu.async_copy(p2.at[wid, pl.ds(q * K, K)], p_v.at[q], psem)

    @pl.loop(0, CH, step=4)
    def _chunk(j):
        for b in range(4):
            jj = j + b

            @pl.when(jj + 3 < CH)
            def _pref():
                pltpu.async_copy(z2p.at[src_v.at[jj + 3]],
                                 rows_v.at[(b + 3) % 4], sem)
                pltpu.async_copy(p2.at[wid, pl.ds((jj + 3) * K, K)],
                                 p_v.at[(b + 3) % 4], psem)

            pltpu.make_async_copy(z2p.at[src_v.at[jj]],
                                  rows_v.at[b], sem).wait()
            pltpu.make_async_copy(p2.at[wid, pl.ds(jj * K, K)],
                                  p_v.at[b], psem).wait()

            @plsc.parallel_loop(0, K, unroll=4)
            def _edge(e):
                sp = p_v[b, e, :]
                for q in range(D2 // 16):
                    sl = pl.ds(q * 16, 16)
                    rows_v[b, e, sl] = rows_v[b, e, sl] * sp

            pltpu.sync_copy(rows_v.at[b], acc.at[dst_v.at[jj]], add=True)

    plsc.subcore_barrier()
    pltpu.sync_copy(acc.at[pl.ds(r0, RPT)], out.at[c, pl.ds(r0, RPT)])


BLK = NP // 8
GRID = NP // BLK


def _rows(width):
    return pl.BlockSpec((BLK, width), lambda i: (i, 0))


def _prows(width):
    return pl.BlockSpec((2, BLK, width), lambda i: (0, i, 0))


def _full(shape):
    return pl.BlockSpec(shape, lambda i: tuple(0 for _ in shape))


def _inv_from_deg(dega_ref):
    deg = dega_ref[0, :, :1] + dega_ref[1, :, :1]
    base = pl.program_id(0) * BLK
    rowmask = base + lax.broadcasted_iota(i32, (BLK, 1), 0) < N
    return jnp.where(jnp.logical_and(rowmask, deg > 0.0),
                     lax.rsqrt(jnp.maximum(deg, 1.0)), 0.0)


def _t1_body(x_ref, br1_ref, s1_ref, dega_ref, za_ref, zb_ref):
    inv = _inv_from_deg(dega_ref)
    wc = jnp.dot(br1_ref[...], s1_ref[...], preferred_element_type=f32)
    z = jnp.dot(x_ref[...], wc, preferred_element_type=f32) * inv
    za_ref[...] = z[:, :DH]
    zb_ref[...] = z[:, DH:]


_t1 = pl.pallas_call(
    _t1_body,
    grid=(GRID,),
    in_specs=[_rows(NFEAT), _full((NFEAT, NBASE * DH1)),
              _full((NBASE * DH1, D1)), _prows(16)],
    out_specs=(_rows(DH), _rows(DH)),
    out_shape=(jax.ShapeDtypeStruct((NP, DH), f32),
               jax.ShapeDtypeStruct((NP, DH), f32)))


def _t2_body(acca_ref, accb_ref, dega_ref, asel_ref, bsel_ref,
             z1a_ref, z1b_ref, a_ref, b_ref):
    inv = _inv_from_deg(dega_ref)
    z1a = (acca_ref[0] + acca_ref[1]) * inv
    z1b = (accb_ref[0] + accb_ref[1]) * inv
    z1a_ref[...] = z1a
    z1b_ref[...] = z1b
    a_ref[...] = (jnp.dot(z1a, asel_ref[:DH, :], preferred_element_type=f32)
                  + jnp.dot(z1b, asel_ref[DH:, :],
                            preferred_element_type=f32))
    b_ref[...] = (jnp.dot(z1a, bsel_ref[:DH, :], preferred_element_type=f32)
                  + jnp.dot(z1b, bsel_ref[DH:, :],
                            preferred_element_type=f32))


_t2 = pl.pallas_call(
    _t2_body,
    grid=(GRID,),
    in_specs=[_prows(DH), _prows(DH), _prows(16),
              _full((D1, 16)), _full((D1, 16))],
    out_specs=(_rows(DH), _rows(DH), _rows(16), _rows(16)),
    out_shape=(jax.ShapeDtypeStruct((NP, DH), f32),
               jax.ShapeDtypeStruct((NP, DH), f32),
               jax.ShapeDtypeStruct((NP, 16), f32),
               jax.ShapeDtypeStruct((NP, 16), f32)))


def _t3_body(outa_ref, outb_ref, outd_ref, dega_ref, bias_ref, br2_ref,
             s2sel_ref, erep_ref, z2pre_ref):
    inv = _inv_from_deg(dega_ref)
    den = outd_ref[0] + outd_ref[1]
    div = jnp.dot(den[:, :H1], erep_ref[...], preferred_element_type=f32)
    num_a = outa_ref[0] + outa_ref[1]
    num_b = outb_ref[0] + outb_ref[1]
    ha = num_a / (div[:, :DH] + 1e-16) + bias_ref[:, :DH]
    hb = num_b / (div[:, DH:] + 1e-16) + bias_ref[:, DH:]
    ha = jnp.where(ha > 0.0, ha, jnp.exp(ha) - 1.0)
    hb = jnp.where(hb > 0.0, hb, jnp.exp(hb) - 1.0)
    wc2 = jnp.dot(br2_ref[...], s2sel_ref[...], preferred_element_type=f32)
    z2 = (jnp.dot(ha, wc2[:DH, :], preferred_element_type=f32)
          + jnp.dot(hb, wc2[DH:, :], preferred_element_type=f32))
    z2pre_ref[...] = z2 * inv


_t3 = pl.pallas_call(
    _t3_body,
    grid=(GRID,),
    in_specs=[_prows(DH), _prows(DH), _prows(16), _prows(16),
              _full((1, D1)), _full((D1, NBASE * D2)),
              _full((NBASE * D2, D2)), _full((H1, D1))],
    out_specs=_rows(D2),
    out_shape=jax.ShapeDtypeStruct((NP, D2), f32))


def _t4_body(acc_ref, dega_ref, a2c_ref, z2p_ref, s2_ref):
    inv = _inv_from_deg(dega_ref)
    z2p = (acc_ref[0] + acc_ref[1]) * inv
    z2p_ref[...] = z2p
    s2_ref[...] = jnp.dot(z2p, a2c_ref[...], preferred_element_type=f32)


_t4 = pl.pallas_call(
    _t4_body,
    grid=(GRID,),
    in_specs=[_prows(D2), _prows(16), _full((D2, 2))],
    out_specs=(_rows(D2), _rows(2)),
    out_shape=(jax.ShapeDtypeStruct((NP, D2), f32),
               jax.ShapeDtypeStruct((NP, 2), f32)))


def _t5_body(outp_ref, outd_ref, fin_ref):
    num = outp_ref[0] + outp_ref[1]
    den = outd_ref[0, :, :1] + outd_ref[1, :, :1]
    fin_ref[...] = num / (den + 1e-16)


_t5 = pl.pallas_call(
    _t5_body,
    grid=(GRID,),
    in_specs=[_prows(D2), _prows(16)],
    out_specs=_rows(D2),
    out_shape=jax.ShapeDtypeStruct((N, D2), f32))


def kernel(x, edge_index, basis1, coef1, a1_src, a1_dst, bias1,
           basis2, coef2, a2_src, a2_dst):
    src = edge_index[0].astype(i32)
    dst = edge_index[1].astype(i32)
    src3 = jnp.concatenate(
        [src.reshape(NW, EW), jnp.zeros((NW, EWP - EW), i32)],
        axis=1).reshape(NW, CH, K)
    dst3 = jnp.concatenate(
        [dst.reshape(NW, EW), jnp.full((NW, EWP - EW), N, i32)],
        axis=1).reshape(NW, CH, K)

    xp = jnp.pad(x, ((0, NP - N), (0, 0)))
    br1 = jnp.transpose(basis1, (1, 0, 2)).reshape(NFEAT, NBASE * DH1)
    s1 = jnp.kron(coef1.T, jnp.eye(DH1, dtype=f32))
    eyeh = jnp.eye(H1, dtype=f32)
    asel_l = (a1_src[:, :, None] * eyeh[:, None, :]).reshape(D1, H1)
    asel_r = (a1_dst[:, :, None] * eyeh[:, None, :]).reshape(D1, H1)
    ta_sel = jnp.concatenate([asel_l, asel_r], axis=1)
    tb_sel = jnp.concatenate([asel_r, asel_l], axis=1)
    br2 = jnp.transpose(basis2, (1, 0, 2)).reshape(D1, NBASE * D2)
    s2sel = jnp.kron(coef2.T, jnp.eye(D2, dtype=f32))
    erep = (eyeh[:, :, None] * jnp.ones((1, 1, DH1), f32)).reshape(H1, D1)
    a2cat = jnp.stack([a2_src[0], a2_dst[0]], axis=1)

    zeros64 = jnp.zeros((NP, DH), f32)
    zeros16 = jnp.zeros((NP, 16), f32)
    ones16 = jnp.ones((K, 16), f32)

    dega = _deg_kernel(dst3, zeros16, ones16)
    zpa, zpb = _t1(xp, br1, s1, dega)
    acca = _prop64(src3, dst3, zpa, zeros64)
    accb = _prop64(src3, dst3, zpb, zeros64)
    z1a, z1b, ta, tb = _t2(acca, accb, dega, ta_sel, tb_sel)
    p1, outd1 = _attp_kernel(src3, dst3, ta, tb, zeros16)
    outa = _att1a(src3, dst3, z1a, p1, zeros64)
    outb = _att1b(src3, dst3, z1b, p1, zeros64)
    z2pre = _t3(outa, outb, outd1, dega, bias1.reshape(1, D1), br2, s2sel,
                erep)
    acc2 = _prop64(src3, dst3, z2pre, zeros64)
    z2p, s2 = _t4(acc2, dega, a2cat)
    es2 = s2[:, 0]
    ed2 = s2[:, 1]
    p2, outd2 = _attp2_kernel(src3, dst3, es2, ed2, zeros16)
    out2 = _att2_kernel(src3, dst3, z2p, p2, zeros64)
    return _t5(out2, outd2)
